# baseline (speedup 1.0000x reference)
import functools

import jax
import jax.numpy as jnp
from jax import lax
from jax.experimental import pallas as pl
from jax.experimental.pallas import tpu as pltpu
from jax.experimental.pallas import tpu_sc as plsc

_L = 16
_K = 5


def _cascade(e, bvec, t, ti):
    c = [e > t[i] for i in range(_K)]
    nt, nti = [], []
    for i in range(_K):
        if i == 0:
            ins_v, ins_i = e, bvec
        else:
            ins_v = jnp.where(c[i - 1], t[i - 1], e)
            ins_i = jnp.where(c[i - 1], ti[i - 1], bvec)
        nt.append(jnp.where(c[i], ins_v, t[i]))
        nti.append(jnp.where(c[i], ins_i, ti[i]))
    return tuple(nt), tuple(nti)


@functools.lru_cache(maxsize=None)
def _build(N, B):
    info = plsc.get_sparse_core_info()
    NS = info.num_subcores
    NW = info.num_cores * NS
    half = B // 2
    row = 3 * B
    groups_total = -(-N // _L)
    gpw = -(-groups_total // NW)
    ppw = gpw * _L
    assert N >= ppw
    mesh = plsc.VectorSubcoreMesh(core_axis_name="c", subcore_axis_name="s")

    out_type = (
        jax.ShapeDtypeStruct((N * 3,), jnp.float32),
        jax.ShapeDtypeStruct((N * 3,), jnp.int32),
        jax.ShapeDtypeStruct((N * 3,), jnp.int32),
        jax.ShapeDtypeStruct((N * _K * 3,), jnp.int32),
        jax.ShapeDtypeStruct((N * _K * 3,), jnp.float32),
    )
    scratch = [
        pltpu.VMEM((_L * row,), jnp.float32),
        pltpu.VMEM((_L * row,), jnp.float32),
        pltpu.VMEM((ppw * 3,), jnp.float32),
        pltpu.VMEM((ppw * 3,), jnp.int32),
        pltpu.VMEM((ppw * 3,), jnp.int32),
        pltpu.VMEM((ppw * _K * 3,), jnp.int32),
        pltpu.VMEM((ppw * _K * 3,), jnp.float32),
    ]

    @functools.partial(
        pl.kernel, out_type=out_type, mesh=mesh, scratch_types=scratch,
        compiler_params=pltpu.CompilerParams(needs_layout_passes=False))
    def launch(vf, gt, o_ps, o_pr, o_gt, o_pt, o_pst,
               fb, gb, ps_v, pr_v, gts_v, pt_v, pst_v):
        wid = lax.axis_index("c") * NS + lax.axis_index("s")
        start = jnp.minimum(wid * ppw, N - ppw)
        lanes = lax.iota(jnp.int32, _L)
        lrow = lanes * row
        l3 = lanes * 3
        l15 = lanes * (3 * _K)

        neg = jnp.full((_L,), -3.4e38, jnp.float32)
        zero = jnp.zeros((_L,), jnp.float32)
        zi = jnp.zeros((_L,), jnp.int32)

        def do_group(g):
            n0 = start + g * _L
            pltpu.sync_copy(vf.at[pl.ds(n0 * row, _L * row)], fb)
            pltpu.sync_copy(gt.at[pl.ds(n0 * row, _L * row)], gb)

            def body1(b, carry):
                m, gm, gi, gs = carry
                bs = jnp.full((_L,), b, jnp.int32)
                i0 = lrow + b * 3
                m_n, gm_n, gi_n, gs_n = [], [], [], []
                for c in range(3):
                    idx = i0 if c == 0 else i0 + c
                    v = plsc.load_gather(fb, [idx])
                    m_n.append(jnp.maximum(m[c], v))
                    w = plsc.load_gather(gb, [idx])
                    cnd = w > gm[c]
                    gm_n.append(jnp.where(cnd, w, gm[c]))
                    gi_n.append(jnp.where(cnd, bs, gi[c]))
                    gs_n.append(gs[c] + w)
                return (tuple(m_n), tuple(gm_n), tuple(gi_n), tuple(gs_n))

            m, gm, gi, gs = lax.fori_loop(
                0, B, body1, ((neg,) * 3, (neg,) * 3, (zi,) * 3, (zero,) * 3))

            def make_body(cas_channels):
                def body(b, carry):
                    s, ts, tis = carry
                    bs = jnp.full((_L,), b, jnp.int32)
                    i0 = lrow + b * 3
                    e = []
                    for c in range(3):
                        idx = i0 if c == 0 else i0 + c
                        v = plsc.load_gather(fb, [idx])
                        e.append(jnp.exp(v - m[c]))
                    s = tuple(s[c] + e[c] for c in range(3))
                    ts = list(ts)
                    tis = list(tis)
                    for c in cas_channels:
                        ts[c], tis[c] = _cascade(e[c], bs, ts[c], tis[c])
                    return (s, tuple(ts), tuple(tis))
                return body

            carry = ((zero,) * 3,
                     tuple((zero,) * _K for _ in range(3)),
                     tuple((zi,) * _K for _ in range(3)))
            carry = lax.fori_loop(0, half, make_body((0, 1)), carry)
            s, ts, tis = lax.fori_loop(half, B, make_body((1, 2)), carry)

            o3 = l3 + (g * _L) * 3
            o15 = l15 + (g * _L) * 3 * _K
            for c in range(3):
                r = 1.0 / s[c]
                oc = o3 if c == 0 else o3 + c
                plsc.store_scatter(ps_v, [oc], ts[c][0] * r)
                plsc.store_scatter(pr_v, [oc], tis[c][0])
                gvals = jnp.where(gs[c] < 0.1,
                                  jnp.full((_L,), -1, jnp.int32), gi[c])
                plsc.store_scatter(gts_v, [oc], gvals)
                for k in range(_K):
                    plsc.store_scatter(pt_v, [o15 + (k * 3 + c)], tis[c][k])
                    plsc.store_scatter(pst_v, [o15 + (k * 3 + c)],
                                       ts[c][k] * r)

        for g in range(gpw):
            do_group(g)

        pltpu.sync_copy(ps_v, o_ps.at[pl.ds(start * 3, ppw * 3)])
        pltpu.sync_copy(pr_v, o_pr.at[pl.ds(start * 3, ppw * 3)])
        pltpu.sync_copy(gts_v, o_gt.at[pl.ds(start * 3, ppw * 3)])
        pltpu.sync_copy(pt_v, o_pt.at[pl.ds(start * 3 * _K, ppw * 3 * _K)])
        pltpu.sync_copy(pst_v, o_pst.at[pl.ds(start * 3 * _K, ppw * 3 * _K)])

    return launch


def kernel(vline_feats, gt_bin, boxes, vps, vert_on, is_roof):
    N, B, C = vline_feats.shape
    launch = _build(N, B)
    ps, pr, gts, pt, pst = launch(
        vline_feats.reshape(N * B * C), gt_bin.reshape(N * B * C))
    return (boxes,
            ps.reshape(N, C),
            pr.reshape(N, C),
            gts.reshape(N, C),
            vps,
            pt.reshape(N, _K, C),
            pst.reshape(N, _K, C))


# speedup vs baseline: 38.6466x; 38.6466x over previous
import functools

import jax
import jax.numpy as jnp
from jax import lax
from jax.experimental import pallas as pl
from jax.experimental.pallas import tpu as pltpu
from jax.experimental.pallas import tpu_sc as plsc

_L = 16
_K = 5
_W = 384


def _cascade(e, bvec, t, ti):
    c = [e > t[i] for i in range(_K)]
    nt, nti = [], []
    for i in range(_K):
        if i == 0:
            ins_v, ins_i = e, bvec
        else:
            ins_v = jnp.where(c[i - 1], t[i - 1], e)
            ins_i = jnp.where(c[i - 1], ti[i - 1], bvec)
        nt.append(jnp.where(c[i], ins_v, t[i]))
        nti.append(jnp.where(c[i], ins_i, ti[i]))
    return tuple(nt), tuple(nti)


@functools.lru_cache(maxsize=None)
def _build(N, B):
    info = plsc.get_sparse_core_info()
    NS = info.num_subcores
    NW = info.num_cores * NS
    half = B // 2
    F = N - N % _L
    gpw = -(-(F // _L) // NW)
    ppw = gpw * _L
    assert F % 128 == 0 and (F - _W) % 128 == 0 and F >= _W >= ppw + 224
    mesh = plsc.VectorSubcoreMesh(core_axis_name="c", subcore_axis_name="s")

    out_type = (
        jax.ShapeDtypeStruct((N * 3,), jnp.float32),
        jax.ShapeDtypeStruct((N * 3,), jnp.int32),
        jax.ShapeDtypeStruct((N * 3,), jnp.int32),
        jax.ShapeDtypeStruct((N * _K * 3,), jnp.int32),
        jax.ShapeDtypeStruct((N * _K * 3,), jnp.float32),
    )
    scratch = [
        pltpu.VMEM((B, _W), jnp.float32),
        pltpu.VMEM((ppw * 3,), jnp.float32),
        pltpu.VMEM((ppw * 3,), jnp.int32),
        pltpu.VMEM((ppw * 3,), jnp.int32),
        pltpu.VMEM((ppw * _K * 3,), jnp.int32),
        pltpu.VMEM((ppw * _K * 3,), jnp.float32),
    ]

    @functools.partial(
        pl.kernel, out_type=out_type, mesh=mesh, scratch_types=scratch,
        compiler_params=pltpu.CompilerParams(needs_layout_passes=False))
    def launch(vf, gt, o_ps, o_pr, o_gt, o_pt, o_pst,
               slab, ps_v, pr_v, gts_v, pt_v, pst_v):
        wid = lax.axis_index("c") * NS + lax.axis_index("s")
        start = jnp.minimum(wid * ppw, F - ppw)
        n_lo = pl.multiple_of(
            jnp.minimum((start // 128) * 128, F - _W), 128)
        off0 = start - n_lo
        lanes = lax.iota(jnp.int32, _L)
        l3 = lanes * 3
        l15 = lanes * (3 * _K)

        zero = jnp.zeros((_L,), jnp.float32)
        zi = jnp.zeros((_L,), jnp.int32)
        neg = jnp.full((_L,), -3.4e38, jnp.float32)

        def bins_loop(lo, hi, unroll, body, init):
            count = hi - lo
            assert count % unroll == 0
            def outer(i, carry):
                b0 = lo + i * unroll
                for u in range(unroll):
                    carry = body(b0 + u, carry)
                return carry
            return lax.fori_loop(0, count // unroll, outer, init)

        def do_subgroup(c, which, off, o3, o15):
            if which == "gt":
                def gbody(b, carry):
                    gm, gi, gs = carry
                    w = slab[b, pl.ds(off, _L)]
                    cnd = w > gm
                    gm = jnp.where(cnd, w, gm)
                    gi = jnp.where(cnd, jnp.full((_L,), b, jnp.int32), gi)
                    return (gm, gi, gs + w)

                gm, gi, gs = bins_loop(0, B, 6, gbody, (neg, zi, zero))
                gvals = jnp.where(gs < 0.1,
                                  jnp.full((_L,), -1, jnp.int32), gi)
                plsc.store_scatter(gts_v, [o3], gvals)
                return

            def mbody(b, m):
                return jnp.maximum(m, slab[b, pl.ds(off, _L)])
            m = bins_loop(0, B, 6, mbody, neg)

            def make_body(cascade_on):
                def body(b, carry):
                    s, t, ti = carry
                    e = jnp.exp(slab[b, pl.ds(off, _L)] - m)
                    s = s + e
                    if cascade_on:
                        t, ti = _cascade(
                            e, jnp.full((_L,), b, jnp.int32), t, ti)
                    return (s, t, ti)
                return body

            carry = (zero, (zero,) * _K, (zi,) * _K)
            lo_cas = c != 2
            hi_cas = c != 0
            carry = bins_loop(0, half, 3 if lo_cas else 6,
                              make_body(lo_cas), carry)
            s, t, ti = bins_loop(half, B, 3 if hi_cas else 6,
                                 make_body(hi_cas), carry)

            r = 1.0 / s
            plsc.store_scatter(ps_v, [o3], t[0] * r)
            plsc.store_scatter(pr_v, [o3], ti[0])
            for k in range(_K):
                plsc.store_scatter(pt_v, [o15 + (k * 3 + c)], ti[k])
                plsc.store_scatter(pst_v, [o15 + (k * 3 + c)], t[k] * r)

        for which in ("feat", "gt"):
            src = vf if which == "feat" else gt
            for c in range(3):
                pltpu.sync_copy(src.at[c, :, pl.ds(n_lo, _W)], slab)
                for j in range(gpw):
                    do_subgroup(c, which,
                                off0 + j * _L,
                                l3 + (j * _L) * 3 + c,
                                l15 + (j * _L) * 3 * _K)

        pltpu.sync_copy(ps_v, o_ps.at[pl.ds(start * 3, ppw * 3)])
        pltpu.sync_copy(pr_v, o_pr.at[pl.ds(start * 3, ppw * 3)])
        pltpu.sync_copy(gts_v, o_gt.at[pl.ds(start * 3, ppw * 3)])
        pltpu.sync_copy(pt_v, o_pt.at[pl.ds(start * 15, ppw * 15)])
        pltpu.sync_copy(pst_v, o_pst.at[pl.ds(start * 15, ppw * 15)])

    return launch


def _masked_prob(vf):
    prob = jax.nn.softmax(vf, axis=1)
    half = vf.shape[1] // 2
    valid = jnp.zeros_like(prob)
    valid = valid.at[:, :half, 0].set(1.0)
    valid = valid.at[:, :, 1].set(1.0)
    valid = valid.at[:, half:, 2].set(1.0)
    return prob * valid


def kernel(vline_feats, gt_bin, boxes, vps, vert_on, is_roof):
    N, B, C = vline_feats.shape
    F = N - N % _L
    launch = _build(N, B)
    vf_t = jnp.transpose(vline_feats, (2, 1, 0))
    gt_t = jnp.transpose(gt_bin, (2, 1, 0))
    ps, pr, gts, pt, pst = launch(vf_t, gt_t)
    ps = ps.reshape(N, C)
    pr = pr.reshape(N, C)
    gts = gts.reshape(N, C)
    pt = pt.reshape(N, _K, C)
    pst = pst.reshape(N, _K, C)

    if F < N:
        p = _masked_prob(vline_feats[F:])
        tg = gt_bin[F:]
        t_gts = jnp.argmax(tg, axis=1)
        t_gts = jnp.where(jnp.sum(tg, axis=1).astype(jnp.float32) < 0.1,
                          -1, t_gts)
        t_sc, t_ix = jax.lax.top_k(jnp.swapaxes(p, 1, 2), _K)
        ps = lax.dynamic_update_slice(ps, jnp.max(p, axis=1), (F, 0))
        pr = lax.dynamic_update_slice(pr, jnp.argmax(p, axis=1), (F, 0))
        gts = lax.dynamic_update_slice(gts, t_gts, (F, 0))
        pt = lax.dynamic_update_slice(pt, jnp.swapaxes(t_ix, 1, 2), (F, 0, 0))
        pst = lax.dynamic_update_slice(pst, jnp.swapaxes(t_sc, 1, 2),
                                       (F, 0, 0))

    return (boxes, ps, pr, gts, vps, pt, pst)


# speedup vs baseline: 45.4608x; 1.1763x over previous
import functools

import jax
import jax.numpy as jnp
from jax import lax
from jax.experimental import pallas as pl
from jax.experimental.pallas import tpu as pltpu
from jax.experimental.pallas import tpu_sc as plsc

_L = 16
_K = 5
_W = 256


def _cascade(e, bvec, t, ti):
    c = [e > t[i] for i in range(_K)]
    nt, nti = [], []
    for i in range(_K):
        if i == 0:
            ins_v, ins_i = e, bvec
        else:
            ins_v = jnp.where(c[i - 1], t[i - 1], e)
            ins_i = jnp.where(c[i - 1], ti[i - 1], bvec)
        nt.append(jnp.where(c[i], ins_v, t[i]))
        nti.append(jnp.where(c[i], ins_i, ti[i]))
    return tuple(nt), tuple(nti)


@functools.lru_cache(maxsize=None)
def _build(N, B):
    info = plsc.get_sparse_core_info()
    NS = info.num_subcores
    NW = info.num_cores * NS
    half = B // 2
    F = N - N % _L
    gpw = -(-(F // _L) // NW)
    ppw = gpw * _L
    assert F % 128 == 0 and (F - _W) % 128 == 0 and F >= _W >= ppw + 96
    assert ppw % 32 == 0 and (F - ppw) % 32 == 0
    mesh = plsc.VectorSubcoreMesh(core_axis_name="c", subcore_axis_name="s")

    out_type = (
        jax.ShapeDtypeStruct((N * 3,), jnp.float32),
        jax.ShapeDtypeStruct((N * 3,), jnp.int32),
        jax.ShapeDtypeStruct((N * 3,), jnp.int32),
        jax.ShapeDtypeStruct((N * _K * 3,), jnp.int32),
        jax.ShapeDtypeStruct((N * _K * 3,), jnp.float32),
    )
    scratch = [
        pltpu.VMEM((B, _W), jnp.float32),
        pltpu.VMEM((B, _W), jnp.float32),
        pltpu.SemaphoreType.DMA,
        pltpu.SemaphoreType.DMA,
        pltpu.VMEM((ppw * 3,), jnp.float32),
        pltpu.VMEM((ppw * 3,), jnp.int32),
        pltpu.VMEM((ppw * 3,), jnp.int32),
        pltpu.VMEM((ppw * _K * 3,), jnp.int32),
        pltpu.VMEM((ppw * _K * 3,), jnp.float32),
    ]

    @functools.partial(
        pl.kernel, out_type=out_type, mesh=mesh, scratch_types=scratch,
        compiler_params=pltpu.CompilerParams(needs_layout_passes=False))
    def launch(vf, gt, o_ps, o_pr, o_gt, o_pt, o_pst,
               slab0, slab1, sem0, sem1, ps_v, pr_v, gts_v, pt_v, pst_v):
        wid = lax.axis_index("c") * NS + lax.axis_index("s")
        start = jnp.minimum(wid * ppw, F - ppw)
        n_lo = pl.multiple_of(
            jnp.minimum((start // 128) * 128, F - _W), 128)
        off0 = start - n_lo
        lanes = lax.iota(jnp.int32, _L)
        l3 = lanes * 3
        l15 = lanes * (3 * _K)

        zero = jnp.zeros((_L,), jnp.float32)
        zi = jnp.zeros((_L,), jnp.int32)
        neg = jnp.full((_L,), -3.4e38, jnp.float32)

        def bins_loop(lo, hi, unroll, body, init):
            count = hi - lo
            assert count % unroll == 0
            def outer(i, carry):
                b0 = lo + i * unroll
                for u in range(unroll):
                    carry = body(b0 + u, carry)
                return carry
            return lax.fori_loop(0, count // unroll, outer, init)

        def do_subgroup(slab, c, which, off, o3, o15):
            if which == "gt":
                def gbody(b, carry):
                    gm, gi, gs = carry
                    w = slab[b, pl.ds(off, _L)]
                    cnd = w > gm
                    gm = jnp.where(cnd, w, gm)
                    gi = jnp.where(cnd, jnp.full((_L,), b, jnp.int32), gi)
                    return (gm, gi, gs + w)

                gm, gi, gs = bins_loop(0, B, 6, gbody, (neg, zi, zero))
                gvals = jnp.where(gs < 0.1,
                                  jnp.full((_L,), -1, jnp.int32), gi)
                plsc.store_scatter(gts_v, [o3], gvals)
                return

            def mbody(b, m):
                return jnp.maximum(m, slab[b, pl.ds(off, _L)])
            m = bins_loop(0, B, 6, mbody, neg)

            def make_body(cascade_on):
                def body(b, carry):
                    s, t, ti = carry
                    e = jnp.exp(slab[b, pl.ds(off, _L)] - m)
                    s = s + e
                    if cascade_on:
                        t, ti = _cascade(
                            e, jnp.full((_L,), b, jnp.int32), t, ti)
                    return (s, t, ti)
                return body

            carry = (zero, (zero,) * _K, (zi,) * _K)
            lo_cas = c != 2
            hi_cas = c != 0
            carry = bins_loop(0, half, 3 if lo_cas else 6,
                              make_body(lo_cas), carry)
            s, t, ti = bins_loop(half, B, 3 if hi_cas else 6,
                                 make_body(hi_cas), carry)

            r = 1.0 / s
            plsc.store_scatter(ps_v, [o3], t[0] * r)
            plsc.store_scatter(pr_v, [o3], ti[0])
            for k in range(_K):
                plsc.store_scatter(pt_v, [o15 + (k * 3 + c)], ti[k])
                plsc.store_scatter(pst_v, [o15 + (k * 3 + c)], t[k] * r)

        windows = [(which, c) for which in ("feat", "gt") for c in range(3)]
        slabs = (slab0, slab1)
        sems = (sem0, sem1)

        def issue(i):
            which, c = windows[i]
            src = vf if which == "feat" else gt
            return pltpu.async_copy(src.at[c, :, pl.ds(n_lo, _W)],
                                    slabs[i % 2], sems[i % 2])

        handle = issue(0)
        for i, (which, c) in enumerate(windows):
            nxt = issue(i + 1) if i + 1 < len(windows) else None
            handle.wait()
            for j in range(gpw):
                do_subgroup(slabs[i % 2], c, which,
                            off0 + j * _L,
                            l3 + (j * _L) * 3 + c,
                            l15 + (j * _L) * 3 * _K)
            handle = nxt

        pltpu.sync_copy(ps_v, o_ps.at[pl.ds(start * 3, ppw * 3)])
        pltpu.sync_copy(pr_v, o_pr.at[pl.ds(start * 3, ppw * 3)])
        pltpu.sync_copy(gts_v, o_gt.at[pl.ds(start * 3, ppw * 3)])
        pltpu.sync_copy(pt_v, o_pt.at[pl.ds(start * 15, ppw * 15)])
        pltpu.sync_copy(pst_v, o_pst.at[pl.ds(start * 15, ppw * 15)])

    return launch


def _masked_prob(vf):
    prob = jax.nn.softmax(vf, axis=1)
    half = vf.shape[1] // 2
    valid = jnp.zeros_like(prob)
    valid = valid.at[:, :half, 0].set(1.0)
    valid = valid.at[:, :, 1].set(1.0)
    valid = valid.at[:, half:, 2].set(1.0)
    return prob * valid


def kernel(vline_feats, gt_bin, boxes, vps, vert_on, is_roof):
    N, B, C = vline_feats.shape
    F = N - N % _L
    launch = _build(N, B)
    vf_t = jnp.transpose(vline_feats, (2, 1, 0))
    gt_t = jnp.transpose(gt_bin, (2, 1, 0))
    ps, pr, gts, pt, pst = launch(vf_t, gt_t)
    ps = ps.reshape(N, C)
    pr = pr.reshape(N, C)
    gts = gts.reshape(N, C)
    pt = pt.reshape(N, _K, C)
    pst = pst.reshape(N, _K, C)

    if F < N:
        p = _masked_prob(vline_feats[F:])
        tg = gt_bin[F:]
        t_gts = jnp.argmax(tg, axis=1)
        t_gts = jnp.where(jnp.sum(tg, axis=1).astype(jnp.float32) < 0.1,
                          -1, t_gts)
        t_sc, t_ix = jax.lax.top_k(jnp.swapaxes(p, 1, 2), _K)
        ps = lax.dynamic_update_slice(ps, jnp.max(p, axis=1), (F, 0))
        pr = lax.dynamic_update_slice(pr, jnp.argmax(p, axis=1), (F, 0))
        gts = lax.dynamic_update_slice(gts, t_gts, (F, 0))
        pt = lax.dynamic_update_slice(pt, jnp.swapaxes(t_ix, 1, 2), (F, 0, 0))
        pst = lax.dynamic_update_slice(pst, jnp.swapaxes(t_sc, 1, 2),
                                       (F, 0, 0))

    return (boxes, ps, pr, gts, vps, pt, pst)
